# baseline (speedup 1.0000x reference)
import functools

import jax
import jax.numpy as jnp
from jax import lax
from jax.experimental import pallas as pl
from jax.experimental.pallas import tpu as pltpu
from jax.experimental.pallas import tpu_sc as plsc

N = 10000
NP = 10240
E = 320000
NTILES = 16
CHUNK = 128
EP = 2560 * CHUNK
NCHUNK = EP // NTILES // CHUNK
IBLK = 16
NBLK = NCHUNK // IBLK
ROWS = NP // NTILES
RCH = 16
F32 = jnp.float32


def _fast_rsqrt(d):
    i = plsc.bitcast(d, jnp.int32)
    i = jnp.int32(0x5F3759DF) - (i >> 1)
    y = plsc.bitcast(i, F32)
    for _ in range(3):
        y = y * (1.5 - 0.5 * d * y * y)
    return y


def _make_sgc(fc, k, first_layer):
    nv = fc // 16
    nbuf = 4 if fc == 64 else 8
    look = nbuf // 2
    mesh = plsc.VectorSubcoreMesh(core_axis_name="c", subcore_axis_name="s")

    def body(refs):
        if first_layer:
            (x_hbm, src_hbm, dst_hbm, out_hbm, norm_hbm,
             G, ACC, DEG, srcb, dstb, *rest) = refs
        else:
            (x_hbm, src_hbm, dst_hbm, normin_hbm, b2_hbm, out_hbm,
             G, ACC, srcb, dstb, *rest) = refs
        srcbB, dstbB = rest[0:2]
        bufs = rest[2:2 + nbuf]
        abuf, gbuf, zbuf, nrm, xtra = rest[2 + nbuf:7 + nbuf]
        ones = bbuf = xtra
        gsems = rest[7 + nbuf:7 + 2 * nbuf]
        ssems = rest[7 + 2 * nbuf:7 + 3 * nbuf]
        isemA, isemB = rest[7 + 3 * nbuf:]
        cid = lax.axis_index("c")
        sid = lax.axis_index("s")
        rowbase = sid * ROWS
        colbase = cid * fc
        zvec = jnp.zeros((16,), F32)

        for r in range(RCH):
            for v in range(nv):
                zbuf[r, pl.ds(v * 16, 16)] = zvec
        if first_layer:
            for i in range(CHUNK // 16):
                ones[pl.ds(i * 16, 16)] = jnp.ones((16,), F32)
            for i in range(ROWS // 16):
                nrm[pl.ds(i * 16, 16)] = zvec
            pltpu.sync_copy(nrm, DEG.at[pl.ds(rowbase, ROWS)])
        else:
            pltpu.sync_copy(normin_hbm.at[pl.ds(rowbase, ROWS)], nrm)
            pltpu.sync_copy(b2_hbm.at[pl.ds(colbase, fc)], bbuf)

        def zacc(c, _):
            pltpu.sync_copy(zbuf, ACC.at[pl.ds(rowbase + c * RCH, RCH), :])
            return 0
        lax.fori_loop(0, ROWS // RCH, zacc, 0)

        if first_layer:
            plsc.subcore_barrier()
            def dblk(b, _):
                base = sid * NCHUNK + b * IBLK
                pltpu.sync_copy(dst_hbm.at[pl.ds(base, IBLK)], dstb)

                def fire(j, _):
                    pltpu.make_async_copy(
                        ones, DEG.at[dstb.at[j]], isemA).start(add=True)
                    return 0
                lax.fori_loop(0, IBLK, fire, 0)

                def drain(j, _):
                    pltpu.make_async_copy(ones, DEG.at[dstb.at[0]], isemA).wait()
                    return 0
                lax.fori_loop(0, IBLK, drain, 0)
                return 0
            lax.fori_loop(0, NBLK, dblk, 0)
            plsc.subcore_barrier()
            pltpu.sync_copy(DEG.at[pl.ds(rowbase, ROWS)], nrm)

            def ncomp(i, _):
                d = jnp.maximum(nrm[pl.ds(i * 16, 16)], 1.0)
                nrm[pl.ds(i * 16, 16)] = _fast_rsqrt(d)
                return 0
            lax.fori_loop(0, ROWS // 16, ncomp, 0)

            @pl.when(cid == 0)
            def _():
                pltpu.sync_copy(nrm, norm_hbm.at[pl.ds(rowbase, ROWS)])

        def g0(c, _):
            rb = rowbase + c * RCH
            pltpu.sync_copy(x_hbm.at[pl.ds(rb, RCH), pl.ds(colbase, fc)], abuf)
            nvec = nrm[pl.ds(c * RCH, RCH)]
            for r in range(RCH):
                s = nvec[r]
                for v in range(nv):
                    sl = pl.ds(v * 16, 16)
                    gbuf[r, sl] = abuf[r, sl] * s
            pltpu.sync_copy(gbuf, G.at[pl.ds(rb, RCH), :])
            return 0
        lax.fori_loop(0, ROWS // RCH, g0, 0)
        plsc.subcore_barrier()

        for rnd in range(k):
            last = rnd == k - 1

            def run_block(blk, sb, db, nsb, ndb):
                nbase = jnp.minimum(
                    sid * NCHUNK + (blk + 1) * IBLK, EP // CHUNK - IBLK)

                @pl.when(blk + 1 < NBLK)
                def _():
                    pltpu.make_async_copy(
                        src_hbm.at[pl.ds(nbase, IBLK)], nsb, isemA).start()
                    pltpu.make_async_copy(
                        dst_hbm.at[pl.ds(nbase, IBLK)], ndb, isemB).start()

                def gath(j, b):
                    return pltpu.make_async_copy(
                        G.at[sb.at[j]], bufs[b], gsems[b])

                def scat(j, b):
                    return pltpu.make_async_copy(
                        bufs[b], ACC.at[db.at[j]], ssems[b])

                for b in range(look):
                    gath(b, b).start()

                def grp(i, _):
                    for b in range(nbuf):
                        j = nbuf * i + b
                        gath(j, b).wait()
                        scat(j, b).start(add=True)
                        nb = (b + look) % nbuf

                        @pl.when(j >= look)
                        def _():
                            scat(j - look, nb).wait()

                        @pl.when(j + look < IBLK)
                        def _():
                            gath(j + look, nb).start()
                    return 0
                lax.fori_loop(0, IBLK // nbuf, grp, 0)
                for jj in range(IBLK - look, IBLK):
                    scat(jj, jj % nbuf).wait()

                @pl.when(blk + 1 < NBLK)
                def _():
                    pltpu.make_async_copy(
                        src_hbm.at[pl.ds(nbase, IBLK)], nsb, isemA).wait()
                    pltpu.make_async_copy(
                        dst_hbm.at[pl.ds(nbase, IBLK)], ndb, isemB).wait()

            base0 = sid * NCHUNK
            pltpu.sync_copy(src_hbm.at[pl.ds(base0, IBLK)], srcb)
            pltpu.sync_copy(dst_hbm.at[pl.ds(base0, IBLK)], dstb)

            def bpair(i, _):
                run_block(2 * i, srcb, dstb, srcbB, dstbB)
                run_block(2 * i + 1, srcbB, dstbB, srcb, dstb)
                return 0
            lax.fori_loop(0, NBLK // 2, bpair, 0)
            plsc.subcore_barrier()

            def spass(c, _):
                rb = rowbase + c * RCH
                pltpu.sync_copy(ACC.at[pl.ds(rb, RCH), :], abuf)
                pltpu.sync_copy(zbuf, ACC.at[pl.ds(rb, RCH), :])
                nvec = nrm[pl.ds(c * RCH, RCH)]
                for r in range(RCH):
                    n = nvec[r]
                    s = n if last else n * n
                    for v in range(nv):
                        sl = pl.ds(v * 16, 16)
                        val = abuf[r, sl] * s
                        if last and not first_layer:
                            val = val + bbuf[sl]
                        gbuf[r, sl] = val
                if last:
                    pltpu.sync_copy(
                        gbuf, out_hbm.at[pl.ds(rb, RCH), pl.ds(colbase, fc)])
                else:
                    pltpu.sync_copy(gbuf, G.at[pl.ds(rb, RCH), :])
                return 0
            lax.fori_loop(0, ROWS // RCH, spass, 0)
            if not last:
                plsc.subcore_barrier()

    f = 2 * fc
    if first_layer:
        out_type = (jax.ShapeDtypeStruct((NP, f), F32),
                    jax.ShapeDtypeStruct((NP,), F32))
        scratch = [
            pltpu.VMEM_SHARED((NP, fc), F32),
            pltpu.VMEM_SHARED((NP, fc), F32),
            pltpu.VMEM_SHARED((NP,), F32),
            pltpu.VMEM((IBLK, CHUNK), jnp.int32),
            pltpu.VMEM((IBLK, CHUNK), jnp.int32),
            pltpu.VMEM((IBLK, CHUNK), jnp.int32),
            pltpu.VMEM((IBLK, CHUNK), jnp.int32),
        ] + [pltpu.VMEM((CHUNK, fc), F32)] * nbuf + [
            pltpu.VMEM((RCH, fc), F32),
            pltpu.VMEM((RCH, fc), F32),
            pltpu.VMEM((RCH, fc), F32),
            pltpu.VMEM((ROWS,), F32),
            pltpu.VMEM((CHUNK,), F32),
        ] + [pltpu.SemaphoreType.DMA] * (2 * nbuf + 2)
    else:
        out_type = jax.ShapeDtypeStruct((NP, f), F32)
        scratch = [
            pltpu.VMEM_SHARED((NP, fc), F32),
            pltpu.VMEM_SHARED((NP, fc), F32),
            pltpu.VMEM((IBLK, CHUNK), jnp.int32),
            pltpu.VMEM((IBLK, CHUNK), jnp.int32),
            pltpu.VMEM((IBLK, CHUNK), jnp.int32),
            pltpu.VMEM((IBLK, CHUNK), jnp.int32),
        ] + [pltpu.VMEM((CHUNK, fc), F32)] * nbuf + [
            pltpu.VMEM((RCH, fc), F32),
            pltpu.VMEM((RCH, fc), F32),
            pltpu.VMEM((RCH, fc), F32),
            pltpu.VMEM((ROWS,), F32),
            pltpu.VMEM((fc,), F32),
        ] + [pltpu.SemaphoreType.DMA] * (2 * nbuf + 2)

    def entry(*args):
        return pl.kernel(
            lambda *refs: body(refs),
            out_type=out_type,
            mesh=mesh,
            scratch_types=scratch,
            compiler_params=pltpu.CompilerParams(
                use_tc_tiling_on_sc=False, needs_layout_passes=False),
        )(*args)

    return entry


_sgc1 = _make_sgc(64, 4, True)
_sgc2 = _make_sgc(32, 4, False)


def _mlp_body(x_ref, w1_ref, b1_ref, w2_ref, o_ref):
    h = jnp.dot(x_ref[...], w1_ref[...], preferred_element_type=F32)
    h = jnp.maximum(h + b1_ref[...], 0.0)
    o_ref[...] = jnp.dot(h, w2_ref[...], preferred_element_type=F32)


def _mlp(f4, W1, b1, W2):
    blk = 1024
    return pl.pallas_call(
        _mlp_body,
        grid=(NP // blk,),
        in_specs=[
            pl.BlockSpec((blk, 128), lambda i: (i, 0)),
            pl.BlockSpec((128, 128), lambda i: (0, 0)),
            pl.BlockSpec((1, 128), lambda i: (0, 0)),
            pl.BlockSpec((128, 64), lambda i: (0, 0)),
        ],
        out_specs=pl.BlockSpec((blk, 64), lambda i: (i, 0)),
        out_shape=jax.ShapeDtypeStruct((NP, 64), F32),
    )(f4, W1, b1.reshape(1, 128), W2)


def kernel(x, edge_index, W1, b1, W2, b2):
    pad_idx = (N + (jnp.arange(EP - E, dtype=jnp.int32) % (NP - N))).astype(jnp.int32)
    src2 = jnp.concatenate([edge_index[0], pad_idx]).reshape(EP // CHUNK, CHUNK)
    dst2 = jnp.concatenate([edge_index[1], pad_idx]).reshape(EP // CHUNK, CHUNK)
    xp = jnp.pad(x, ((0, NP - N), (0, 0)))
    f4, norm = _sgc1(xp, src2, dst2)
    z = _mlp(f4, W1, b1, W2)
    f8 = _sgc2(z, src2, dst2, norm, b2)
    return f8[:N]


# speedup vs baseline: 1.0969x; 1.0969x over previous
import functools

import jax
import jax.numpy as jnp
from jax import lax
from jax.experimental import pallas as pl
from jax.experimental.pallas import tpu as pltpu
from jax.experimental.pallas import tpu_sc as plsc

N = 10000
NP = 10240
E = 320000
NTILES = 16
CHUNK = 128
EP = 2560 * CHUNK
NCHUNK = EP // NTILES // CHUNK
IBLK = 32
NBLK = NCHUNK // IBLK
ROWS = NP // NTILES
RCH = 16
F32 = jnp.float32


def _fast_rsqrt(d):
    i = plsc.bitcast(d, jnp.int32)
    i = jnp.int32(0x5F3759DF) - (i >> 1)
    y = plsc.bitcast(i, F32)
    for _ in range(3):
        y = y * (1.5 - 0.5 * d * y * y)
    return y


def _make_sgc(fc, k, first_layer):
    nv = fc // 16
    nbuf = 4 if fc == 64 else 8
    look = nbuf // 2
    mesh = plsc.VectorSubcoreMesh(core_axis_name="c", subcore_axis_name="s")

    def body(refs):
        if first_layer:
            (x_hbm, src_hbm, dst_hbm, out_hbm, norm_hbm,
             G, ACC, DEG, srcb, dstb, *rest) = refs
        else:
            (x_hbm, src_hbm, dst_hbm, normin_hbm, b2_hbm, out_hbm,
             G, ACC, srcb, dstb, *rest) = refs
        bufs = rest[:nbuf]
        abuf, gbuf, zbuf, nrm, xtra = rest[nbuf:nbuf + 5]
        ones = bbuf = xtra
        gsems = rest[nbuf + 5:nbuf + 5 + nbuf]
        ssems = rest[nbuf + 5 + nbuf:nbuf + 5 + 2 * nbuf]
        isemA = rest[nbuf + 5 + 2 * nbuf]
        cid = lax.axis_index("c")
        sid = lax.axis_index("s")
        rowbase = sid * ROWS
        colbase = cid * fc
        zvec = jnp.zeros((16,), F32)

        for r in range(RCH):
            for v in range(nv):
                zbuf[r, pl.ds(v * 16, 16)] = zvec
        if first_layer:
            for i in range(CHUNK // 16):
                ones[pl.ds(i * 16, 16)] = jnp.ones((16,), F32)
            for i in range(ROWS // 16):
                nrm[pl.ds(i * 16, 16)] = zvec
            pltpu.sync_copy(nrm, DEG.at[pl.ds(rowbase, ROWS)])
        else:
            pltpu.sync_copy(normin_hbm.at[pl.ds(rowbase, ROWS)], nrm)
            pltpu.sync_copy(b2_hbm.at[pl.ds(colbase, fc)], bbuf)

        def zacc(c, _):
            pltpu.sync_copy(zbuf, ACC.at[pl.ds(rowbase + c * RCH, RCH), :])
            return 0
        lax.fori_loop(0, ROWS // RCH, zacc, 0)

        if first_layer:
            plsc.subcore_barrier()
            def dblk(b, _):
                base = sid * NCHUNK + b * IBLK
                pltpu.sync_copy(dst_hbm.at[pl.ds(base, IBLK)], dstb)

                def fire(j, _):
                    pltpu.make_async_copy(
                        ones, DEG.at[dstb.at[j]], isemA).start(add=True)
                    return 0
                lax.fori_loop(0, IBLK, fire, 0)

                def drain(j, _):
                    pltpu.make_async_copy(ones, DEG.at[dstb.at[0]], isemA).wait()
                    return 0
                lax.fori_loop(0, IBLK, drain, 0)
                return 0
            lax.fori_loop(0, NBLK, dblk, 0)
            plsc.subcore_barrier()
            pltpu.sync_copy(DEG.at[pl.ds(rowbase, ROWS)], nrm)

            def ncomp(i, _):
                d = jnp.maximum(nrm[pl.ds(i * 16, 16)], 1.0)
                nrm[pl.ds(i * 16, 16)] = _fast_rsqrt(d)
                return 0
            lax.fori_loop(0, ROWS // 16, ncomp, 0)

            @pl.when(cid == 0)
            def _():
                pltpu.sync_copy(nrm, norm_hbm.at[pl.ds(rowbase, ROWS)])

        def g0(c, _):
            rb = rowbase + c * RCH
            pltpu.sync_copy(x_hbm.at[pl.ds(rb, RCH), pl.ds(colbase, fc)], abuf)
            nvec = nrm[pl.ds(c * RCH, RCH)]
            for r in range(RCH):
                s = nvec[r]
                for v in range(nv):
                    sl = pl.ds(v * 16, 16)
                    gbuf[r, sl] = abuf[r, sl] * s
            pltpu.sync_copy(gbuf, G.at[pl.ds(rb, RCH), :])
            return 0
        lax.fori_loop(0, ROWS // RCH, g0, 0)
        plsc.subcore_barrier()

        for rnd in range(k):
            last = rnd == k - 1

            def gath(j, b):
                return pltpu.make_async_copy(G.at[srcb.at[j]], bufs[b], gsems[b])

            def scat(j, b):
                return pltpu.make_async_copy(bufs[b], ACC.at[dstb.at[j]], ssems[b])

            def eblock(blk, _):
                base = sid * NCHUNK + blk * IBLK
                pltpu.sync_copy(src_hbm.at[pl.ds(base, IBLK)], srcb)
                pltpu.sync_copy(dst_hbm.at[pl.ds(base, IBLK)], dstb)
                for b in range(look):
                    gath(b, b).start()

                def grp(i, _):
                    for b in range(nbuf):
                        j = nbuf * i + b
                        gath(j, b).wait()
                        scat(j, b).start(add=True)
                        nb = (b + look) % nbuf

                        @pl.when(j >= look)
                        def _():
                            scat(j - look, nb).wait()

                        @pl.when(j + look < IBLK)
                        def _():
                            gath(j + look, nb).start()
                    return 0
                lax.fori_loop(0, IBLK // nbuf, grp, 0)
                for jj in range(IBLK - look, IBLK):
                    scat(jj, jj % nbuf).wait()
                return 0
            lax.fori_loop(0, NBLK, eblock, 0)
            plsc.subcore_barrier()

            def spass(c, _):
                rb = rowbase + c * RCH
                pltpu.sync_copy(ACC.at[pl.ds(rb, RCH), :], abuf)
                pltpu.sync_copy(zbuf, ACC.at[pl.ds(rb, RCH), :])
                nvec = nrm[pl.ds(c * RCH, RCH)]
                for r in range(RCH):
                    n = nvec[r]
                    s = n if last else n * n
                    for v in range(nv):
                        sl = pl.ds(v * 16, 16)
                        val = abuf[r, sl] * s
                        if last and not first_layer:
                            val = val + bbuf[sl]
                        gbuf[r, sl] = val
                if last:
                    pltpu.sync_copy(
                        gbuf, out_hbm.at[pl.ds(rb, RCH), pl.ds(colbase, fc)])
                else:
                    pltpu.sync_copy(gbuf, G.at[pl.ds(rb, RCH), :])
                return 0
            lax.fori_loop(0, ROWS // RCH, spass, 0)
            if not last:
                plsc.subcore_barrier()

    f = 2 * fc
    if first_layer:
        out_type = (jax.ShapeDtypeStruct((NP, f), F32),
                    jax.ShapeDtypeStruct((NP,), F32))
        scratch = [
            pltpu.VMEM_SHARED((NP, fc), F32),
            pltpu.VMEM_SHARED((NP, fc), F32),
            pltpu.VMEM_SHARED((NP,), F32),
            pltpu.VMEM((IBLK, CHUNK), jnp.int32),
            pltpu.VMEM((IBLK, CHUNK), jnp.int32),
        ] + [pltpu.VMEM((CHUNK, fc), F32)] * nbuf + [
            pltpu.VMEM((RCH, fc), F32),
            pltpu.VMEM((RCH, fc), F32),
            pltpu.VMEM((RCH, fc), F32),
            pltpu.VMEM((ROWS,), F32),
            pltpu.VMEM((CHUNK,), F32),
        ] + [pltpu.SemaphoreType.DMA] * (2 * nbuf + 1)
    else:
        out_type = jax.ShapeDtypeStruct((NP, f), F32)
        scratch = [
            pltpu.VMEM_SHARED((NP, fc), F32),
            pltpu.VMEM_SHARED((NP, fc), F32),
            pltpu.VMEM((IBLK, CHUNK), jnp.int32),
            pltpu.VMEM((IBLK, CHUNK), jnp.int32),
        ] + [pltpu.VMEM((CHUNK, fc), F32)] * nbuf + [
            pltpu.VMEM((RCH, fc), F32),
            pltpu.VMEM((RCH, fc), F32),
            pltpu.VMEM((RCH, fc), F32),
            pltpu.VMEM((ROWS,), F32),
            pltpu.VMEM((fc,), F32),
        ] + [pltpu.SemaphoreType.DMA] * (2 * nbuf + 1)

    def entry(*args):
        return pl.kernel(
            lambda *refs: body(refs),
            out_type=out_type,
            mesh=mesh,
            scratch_types=scratch,
            compiler_params=pltpu.CompilerParams(
                use_tc_tiling_on_sc=False, needs_layout_passes=False),
        )(*args)

    return entry


_sgc1 = _make_sgc(64, 4, True)
_sgc2 = _make_sgc(32, 4, False)


def _mlp_body(x_ref, w1_ref, b1_ref, w2_ref, o_ref):
    h = jnp.dot(x_ref[...], w1_ref[...], preferred_element_type=F32)
    h = jnp.maximum(h + b1_ref[...], 0.0)
    o_ref[...] = jnp.dot(h, w2_ref[...], preferred_element_type=F32)


def _mlp(f4, W1, b1, W2):
    blk = 1024
    return pl.pallas_call(
        _mlp_body,
        grid=(NP // blk,),
        in_specs=[
            pl.BlockSpec((blk, 128), lambda i: (i, 0)),
            pl.BlockSpec((128, 128), lambda i: (0, 0)),
            pl.BlockSpec((1, 128), lambda i: (0, 0)),
            pl.BlockSpec((128, 64), lambda i: (0, 0)),
        ],
        out_specs=pl.BlockSpec((blk, 64), lambda i: (i, 0)),
        out_shape=jax.ShapeDtypeStruct((NP, 64), F32),
    )(f4, W1, b1.reshape(1, 128), W2)


def kernel(x, edge_index, W1, b1, W2, b2):
    pad_idx = (N + (jnp.arange(EP - E, dtype=jnp.int32) % (NP - N))).astype(jnp.int32)
    src2 = jnp.concatenate([edge_index[0], pad_idx]).reshape(EP // CHUNK, CHUNK)
    dst2 = jnp.concatenate([edge_index[1], pad_idx]).reshape(EP // CHUNK, CHUNK)
    xp = jnp.pad(x, ((0, NP - N), (0, 0)))
    f4, norm = _sgc1(xp, src2, dst2)
    z = _mlp(f4, W1, b1, W2)
    f8 = _sgc2(z, src2, dst2, norm, b2)
    return f8[:N]


# speedup vs baseline: 1.2009x; 1.0948x over previous
import functools

import jax
import jax.numpy as jnp
from jax import lax
from jax.experimental import pallas as pl
from jax.experimental.pallas import tpu as pltpu
from jax.experimental.pallas import tpu_sc as plsc

N = 10000
NP = 10240
E = 320000
NTILES = 16
CHUNK = 128
EP = 2560 * CHUNK
NCHUNK = EP // NTILES // CHUNK
IBLK = 32
NBLK = NCHUNK // IBLK
ROWS = NP // NTILES
F32 = jnp.float32


def _fast_rsqrt(d):
    i = plsc.bitcast(d, jnp.int32)
    i = jnp.int32(0x5F3759DF) - (i >> 1)
    y = plsc.bitcast(i, F32)
    for _ in range(3):
        y = y * (1.5 - 0.5 * d * y * y)
    return y


def _make_sgc(fc, k, first_layer):
    nv = fc // 16
    nbuf = 4 if fc == 64 else 8
    look = nbuf // 2
    rch = 16 if fc == 64 else 32
    nch = ROWS // rch
    mesh = plsc.VectorSubcoreMesh(core_axis_name="c", subcore_axis_name="s")

    def body(refs):
        if first_layer:
            (x_hbm, src_hbm, dst_hbm, out_hbm, norm_hbm,
             G, ACC, DEG, srcb, dstb, *rest) = refs
        else:
            (x_hbm, src_hbm, dst_hbm, normin_hbm, b2_hbm, out_hbm,
             G, ACC, srcb, dstb, *rest) = refs
        bufs = rest[:nbuf]
        abufs = rest[nbuf:nbuf + 2]
        gbufs = rest[nbuf + 2:nbuf + 4]
        zbuf, nrm, xtra = rest[nbuf + 4:nbuf + 7]
        ones = bbuf = xtra
        gsems = rest[nbuf + 7:nbuf + 7 + nbuf]
        ssems = rest[nbuf + 7 + nbuf:nbuf + 7 + 2 * nbuf]
        asems = rest[nbuf + 7 + 2 * nbuf:nbuf + 9 + 2 * nbuf]
        osems = rest[nbuf + 9 + 2 * nbuf:nbuf + 11 + 2 * nbuf]
        zsems = rest[nbuf + 11 + 2 * nbuf:nbuf + 13 + 2 * nbuf]
        cid = lax.axis_index("c")
        sid = lax.axis_index("s")
        rowbase = sid * ROWS
        colbase = cid * fc
        zvec = jnp.zeros((16,), F32)

        for r in range(rch):
            for v in range(nv):
                zbuf[r, pl.ds(v * 16, 16)] = zvec
        if first_layer:
            for i in range(CHUNK // 16):
                ones[pl.ds(i * 16, 16)] = jnp.ones((16,), F32)
            for i in range(ROWS // 16):
                nrm[pl.ds(i * 16, 16)] = zvec
            pltpu.sync_copy(nrm, DEG.at[pl.ds(rowbase, ROWS)])
        else:
            pltpu.sync_copy(normin_hbm.at[pl.ds(rowbase, ROWS)], nrm)
            pltpu.sync_copy(b2_hbm.at[pl.ds(colbase, fc)], bbuf)

        def zacc(c, _):
            pltpu.sync_copy(zbuf, ACC.at[pl.ds(rowbase + c * rch, rch), :])
            return 0
        lax.fori_loop(0, nch, zacc, 0)

        if first_layer:
            plsc.subcore_barrier()
            def dblk(b, _):
                base = sid * NCHUNK + b * IBLK
                pltpu.sync_copy(dst_hbm.at[pl.ds(base, IBLK)], dstb)

                def fire(j, _):
                    pltpu.make_async_copy(
                        ones, DEG.at[dstb.at[j]], asems[0]).start(add=True)
                    return 0
                lax.fori_loop(0, IBLK, fire, 0)

                def drain(j, _):
                    pltpu.make_async_copy(
                        ones, DEG.at[dstb.at[0]], asems[0]).wait()
                    return 0
                lax.fori_loop(0, IBLK, drain, 0)
                return 0
            lax.fori_loop(0, NBLK, dblk, 0)
            plsc.subcore_barrier()
            pltpu.sync_copy(DEG.at[pl.ds(rowbase, ROWS)], nrm)

            def ncomp(i, _):
                d = jnp.maximum(nrm[pl.ds(i * 16, 16)], 1.0)
                nrm[pl.ds(i * 16, 16)] = _fast_rsqrt(d)
                return 0
            lax.fori_loop(0, ROWS // 16, ncomp, 0)

            @pl.when(cid == 0)
            def _():
                pltpu.sync_copy(nrm, norm_hbm.at[pl.ds(rowbase, ROWS)])

        def scale_chunk(c, ab, gb, power1, add_bias):
            nvecs = [nrm[pl.ds(c * rch + 16 * q, 16)] for q in range(rch // 16)]
            for r in range(rch):
                n = nvecs[r // 16][r % 16]
                s = n if power1 else n * n
                for v in range(nv):
                    sl = pl.ds(v * 16, 16)
                    val = ab[r, sl] * s
                    if add_bias:
                        val = val + bbuf[sl]
                    gb[r, sl] = val

        def g0(c, _):
            rb = rowbase + c * rch
            pltpu.sync_copy(x_hbm.at[pl.ds(rb, rch), pl.ds(colbase, fc)],
                            abufs[0])
            scale_chunk(c, abufs[0], gbufs[0], True, False)
            pltpu.sync_copy(gbufs[0], G.at[pl.ds(rb, rch), :])
            return 0
        lax.fori_loop(0, nch, g0, 0)
        plsc.subcore_barrier()

        def spass_round(last):
            def pre(c, p):
                return pltpu.make_async_copy(
                    ACC.at[pl.ds(rowbase + c * rch, rch), :], abufs[p],
                    asems[p])

            def gout(c, p):
                if last:
                    dst = out_hbm.at[pl.ds(rowbase + c * rch, rch),
                                     pl.ds(colbase, fc)]
                else:
                    dst = G.at[pl.ds(rowbase + c * rch, rch), :]
                return pltpu.make_async_copy(gbufs[p], dst, osems[p])

            def zout(c, p):
                return pltpu.make_async_copy(
                    zbuf, ACC.at[pl.ds(rowbase + c * rch, rch), :], zsems[p])

            pre(0, 0).start()

            def duo(i, _):
                for p in range(2):
                    c = 2 * i + p
                    pre(c, p).wait()

                    @pl.when(c + 1 < nch)
                    def _():
                        pre(c + 1, 1 - p).start()

                    @pl.when(c >= 2)
                    def _():
                        gout(c - 2, p).wait()
                        zout(c - 2, p).wait()
                    scale_chunk(c, abufs[p], gbufs[p], last,
                                last and not first_layer)
                    gout(c, p).start()
                    zout(c, p).start()
                return 0
            lax.fori_loop(0, nch // 2, duo, 0)
            for cc in (nch - 2, nch - 1):
                gout(cc, cc % 2).wait()
                zout(cc, cc % 2).wait()

        def gath(j, b):
            return pltpu.make_async_copy(G.at[srcb.at[j]], bufs[b], gsems[b])

        def scat(j, b):
            return pltpu.make_async_copy(bufs[b], ACC.at[dstb.at[j]], ssems[b])

        def eblock(blk, _):
            base = sid * NCHUNK + blk * IBLK
            pltpu.sync_copy(src_hbm.at[pl.ds(base, IBLK)], srcb)
            pltpu.sync_copy(dst_hbm.at[pl.ds(base, IBLK)], dstb)
            for b in range(look):
                gath(b, b).start()

            def grp(i, _):
                for b in range(nbuf):
                    j = nbuf * i + b
                    gath(j, b).wait()
                    scat(j, b).start(add=True)
                    nb = (b + look) % nbuf

                    @pl.when(j >= look)
                    def _():
                        scat(j - look, nb).wait()

                    @pl.when(j + look < IBLK)
                    def _():
                        gath(j + look, nb).start()
                return 0
            lax.fori_loop(0, IBLK // nbuf, grp, 0)
            for jj in range(IBLK - look, IBLK):
                scat(jj, jj % nbuf).wait()
            return 0

        for rnd in range(k):
            lax.fori_loop(0, NBLK, eblock, 0)
            plsc.subcore_barrier()
            spass_round(rnd == k - 1)
            if rnd < k - 1:
                plsc.subcore_barrier()

    f = 2 * fc
    if first_layer:
        out_type = (jax.ShapeDtypeStruct((NP, f), F32),
                    jax.ShapeDtypeStruct((NP,), F32))
        scratch = [
            pltpu.VMEM_SHARED((NP, fc), F32),
            pltpu.VMEM_SHARED((NP, fc), F32),
            pltpu.VMEM_SHARED((NP,), F32),
        ]
        xtra_shape = (CHUNK,)
    else:
        out_type = jax.ShapeDtypeStruct((NP, f), F32)
        scratch = [
            pltpu.VMEM_SHARED((NP, fc), F32),
            pltpu.VMEM_SHARED((NP, fc), F32),
        ]
        xtra_shape = (fc,)
    scratch += [
        pltpu.VMEM((IBLK, CHUNK), jnp.int32),
        pltpu.VMEM((IBLK, CHUNK), jnp.int32),
    ] + [pltpu.VMEM((CHUNK, fc), F32)] * nbuf + [
        pltpu.VMEM((rch, fc), F32),
        pltpu.VMEM((rch, fc), F32),
        pltpu.VMEM((rch, fc), F32),
        pltpu.VMEM((rch, fc), F32),
        pltpu.VMEM((rch, fc), F32),
        pltpu.VMEM((ROWS,), F32),
        pltpu.VMEM(xtra_shape, F32),
    ] + [pltpu.SemaphoreType.DMA] * (2 * nbuf + 6)

    def entry(*args):
        return pl.kernel(
            lambda *refs: body(refs),
            out_type=out_type,
            mesh=mesh,
            scratch_types=scratch,
            compiler_params=pltpu.CompilerParams(
                use_tc_tiling_on_sc=False, needs_layout_passes=False),
        )(*args)

    return entry


_sgc1 = _make_sgc(64, 4, True)
_sgc2 = _make_sgc(32, 4, False)


def _mlp_body(x_ref, w1_ref, b1_ref, w2_ref, o_ref):
    h = jnp.dot(x_ref[...], w1_ref[...], preferred_element_type=F32)
    h = jnp.maximum(h + b1_ref[...], 0.0)
    o_ref[...] = jnp.dot(h, w2_ref[...], preferred_element_type=F32)


def _mlp(f4, W1, b1, W2):
    blk = 1024
    return pl.pallas_call(
        _mlp_body,
        grid=(NP // blk,),
        in_specs=[
            pl.BlockSpec((blk, 128), lambda i: (i, 0)),
            pl.BlockSpec((128, 128), lambda i: (0, 0)),
            pl.BlockSpec((1, 128), lambda i: (0, 0)),
            pl.BlockSpec((128, 64), lambda i: (0, 0)),
        ],
        out_specs=pl.BlockSpec((blk, 64), lambda i: (i, 0)),
        out_shape=jax.ShapeDtypeStruct((NP, 64), F32),
    )(f4, W1, b1.reshape(1, 128), W2)


def kernel(x, edge_index, W1, b1, W2, b2):
    pad_idx = (N + (jnp.arange(EP - E, dtype=jnp.int32) % (NP - N))).astype(jnp.int32)
    src2 = jnp.concatenate([edge_index[0], pad_idx]).reshape(EP // CHUNK, CHUNK)
    dst2 = jnp.concatenate([edge_index[1], pad_idx]).reshape(EP // CHUNK, CHUNK)
    xp = jnp.pad(x, ((0, NP - N), (0, 0)))
    f4, norm = _sgc1(xp, src2, dst2)
    z = _mlp(f4, W1, b1, W2)
    f8 = _sgc2(z, src2, dst2, norm, b2)
    return f8[:N]
